# baseline (speedup 1.0000x reference)
import functools

import jax
import jax.numpy as jnp
import numpy as np
from jax import lax
from jax.experimental import pallas as pl
from jax.experimental.pallas import tpu as pltpu
from jax.experimental.pallas import tpu_sc as plsc

_THR = 0.15
N_IN = 2048
N_OUT = 2048
K = 9
C = 256
NC, NS, L = 2, 16, 16
NW = NC * NS
OPW = N_OUT // NW
MAXC = 192
NCH = N_IN // L
CCH = MAXC // L


def _sqrt_newton(a, iters=28):
    y = jnp.float32(1.0)
    for _ in range(iters):
        y = y * (1.5 - 0.5 * a * y * y)
    return a * y


def _round_bf16(s):
    r = lax.bitcast_convert_type(s, jnp.int32)
    r = r + jnp.int32(0x7FFF) + lax.shift_right_logical(r, 16) % 2
    r = r & jnp.int32(-65536)
    return lax.bitcast_convert_type(r, jnp.float32)


def _recip_newton(x, iters=16):
    r = jnp.float32(1.0 / 512.0)
    for _ in range(iters):
        r = r * (2.0 - x * r)
    return r


def _sc_body(x_hbm, qinf_hbm, qoutf_hbm, qkerf_hbm, agg_hbm,
             qraw_v, qx_v, qy_v, qz_v, qw_v, qoutf_v, qkerf_v,
             cand_v, taprow_v, candx_v, aggrow_v, sem):
    wid = lax.axis_index("s") * NC + lax.axis_index("c")

    pltpu.sync_copy(qinf_hbm, qraw_v)
    pltpu.sync_copy(qoutf_hbm.at[pl.ds(wid * OPW * 4, OPW * 4)],
                    qoutf_v.at[pl.ds(0, OPW * 4)])
    pltpu.sync_copy(qkerf_hbm, qkerf_v.at[pl.ds(0, K * 4)])

    lane = lax.broadcasted_iota(jnp.int32, (L,), 0)

    def deinter(c, _):
        base = (c * L + lane) * 4
        qx_v[pl.ds(c * L, L)] = _round_bf16(plsc.load_gather(qraw_v, [base]))
        qy_v[pl.ds(c * L, L)] = _round_bf16(
            plsc.load_gather(qraw_v, [base + 1]))
        qz_v[pl.ds(c * L, L)] = _round_bf16(
            plsc.load_gather(qraw_v, [base + 2]))
        qw_v[pl.ds(c * L, L)] = _round_bf16(
            plsc.load_gather(qraw_v, [base + 3]))
        return 0

    lax.fori_loop(0, NCH, deinter, 0)

    zero16 = jnp.zeros((L,), jnp.int32)
    for c in range(CCH + 1):
        cand_v[pl.ds(c * L, L)] = zero16

    ch = jnp.float32(np.cos(_THR / 2.0))
    kq = []
    minw = jnp.float32(1.0)
    for k in range(K):
        kv = qkerf_v[pl.ds(4 * k, L)]
        kq.append((kv[0], kv[1], kv[2], kv[3]))
        minw = jnp.minimum(minw, jnp.abs(kv[3]))
    cos_a = ch - jnp.float32(6e-3)
    sin_a = _sqrt_newton(jnp.maximum(1.0 - cos_a * cos_a, 0.0)) \
        + jnp.float32(1e-4)
    sb = _sqrt_newton(jnp.maximum(1.0 - minw * minw, 0.0)) + jnp.float32(1e-4)
    cos_cand = cos_a * minw - sin_a * sb - jnp.float32(3e-3)

    def per_output(o, _):
        ov = qoutf_v[pl.ds(4 * o, L)]
        ox, oy, oz, ow = ov[0], ov[1], ov[2], ov[3]

        def scan_chunk(c, cnt):
            qx = qx_v[pl.ds(c * L, L)]
            qy = qy_v[pl.ds(c * L, L)]
            qz = qz_v[pl.ds(c * L, L)]
            qw = qw_v[pl.ds(c * L, L)]
            d = ox * qx + oy * qy + oz * qz + ow * qw
            m = jnp.abs(d) > cos_cand
            off = jnp.minimum(cnt, MAXC)
            cum = plsc.cumsum(jnp.where(m, jnp.int32(1), jnp.int32(0)))
            plsc.store_scatter(cand_v, [off + cum - 1], c * L + lane, mask=m)
            return cnt + cum[L - 1]

        cnt = lax.fori_loop(0, NCH, scan_chunk, jnp.int32(0))
        cnt = jnp.minimum(cnt, MAXC)

        pltpu.async_copy(x_hbm.at[cand_v.at[pl.ds(0, MAXC)]], candx_v,
                         sem).wait()

        for k in range(K):
            kx, ky, kz, kw = kq[k]
            ix = _round_bf16(ow * kx + ox * kw + oy * kz - oz * ky)
            iy = _round_bf16(ow * ky - ox * kz + oy * kw + oz * kx)
            iz = _round_bf16(ow * kz + ox * ky - oy * kx + oz * kw)
            iw = _round_bf16(ow * kw - ox * kx - oy * ky - oz * kz)

            def tap_chunk(c, cntk):
                idx = cand_v[pl.ds(c * L, L)]
                valid = (c * L + lane) < cnt
                gx = plsc.load_gather(qx_v, [idx], mask=valid)
                gy = plsc.load_gather(qy_v, [idx], mask=valid)
                gz = plsc.load_gather(qz_v, [idx], mask=valid)
                gw = plsc.load_gather(qw_v, [idx], mask=valid)
                d = ix * gx + iy * gy + iz * gz + iw * gw
                am = (jnp.abs(d) > ch) & valid
                cum = plsc.cumsum(jnp.where(am, jnp.int32(1), jnp.int32(0)))
                plsc.store_scatter(taprow_v, [cntk + cum - 1], c * L + lane,
                                   mask=am)
                return cntk + cum[L - 1]

            cntk = lax.fori_loop(0, CCH, tap_chunk, jnp.int32(0))

            def add_row(r, acc):
                rv = taprow_v[pl.ds(r, L)]
                pos = rv[0]
                return tuple(
                    acc[c2] + candx_v[pos, pl.ds(c2 * L, L)]
                    for c2 in range(C // L))

            acc0 = tuple(jnp.zeros((L,), jnp.float32)
                         for _ in range(C // L))
            acc = lax.fori_loop(0, cntk, add_row, acc0)
            inv = _recip_newton(jnp.maximum(cntk.astype(jnp.float32), 1.0))
            for c2 in range(C // L):
                aggrow_v[pl.ds(k * C + c2 * L, L)] = acc[c2] * inv

        go = wid * OPW + o
        pltpu.sync_copy(aggrow_v, agg_hbm.at[go])
        return 0

    lax.fori_loop(0, OPW, per_output, 0)


def _sc_stage(x, q_in_flat, q_out_flat, q_ker_flat):
    kern = pl.kernel(
        _sc_body,
        out_type=jax.ShapeDtypeStruct((N_OUT, K * C), jnp.float32),
        mesh=plsc.VectorSubcoreMesh(core_axis_name="c", subcore_axis_name="s"),
        compiler_params=pltpu.CompilerParams(needs_layout_passes=False,
                                             use_tc_tiling_on_sc=True),
        scratch_types=[
            pltpu.VMEM((N_IN * 4,), jnp.float32),
            pltpu.VMEM((N_IN,), jnp.float32),
            pltpu.VMEM((N_IN,), jnp.float32),
            pltpu.VMEM((N_IN,), jnp.float32),
            pltpu.VMEM((N_IN,), jnp.float32),
            pltpu.VMEM((OPW * 4 + L,), jnp.float32),
            pltpu.VMEM((K * 4 + L,), jnp.float32),
            pltpu.VMEM((MAXC + L,), jnp.int32),
            pltpu.VMEM((MAXC + L,), jnp.int32),
            pltpu.VMEM((MAXC, C), jnp.float32),
            pltpu.VMEM((K * C,), jnp.float32),
            pltpu.SemaphoreType.DMA,
        ],
    )
    return kern(x, q_in_flat, q_out_flat, q_ker_flat)


def _tc_body(agg_ref, wf_ref, b_ref, out_ref):
    out_ref[...] = (jnp.dot(agg_ref[...], wf_ref[...],
                            preferred_element_type=jnp.float32)
                    + b_ref[...])


def _tc_stage(agg, w_flat, b):
    block_o = 256
    return pl.pallas_call(
        _tc_body,
        grid=(N_OUT // block_o,),
        in_specs=[
            pl.BlockSpec((block_o, K * C), lambda i: (i, 0)),
            pl.BlockSpec((K * C, C), lambda i: (0, 0)),
            pl.BlockSpec((1, C), lambda i: (0, 0)),
        ],
        out_specs=pl.BlockSpec((block_o, C), lambda i: (i, 0)),
        out_shape=jax.ShapeDtypeStruct((N_OUT, C), jnp.float32),
    )(agg, w_flat, b.reshape(1, C))


@functools.partial(jax.jit, static_argnames=())
def kernel(input_features, W, b, q_in, q_out, q_ker):
    agg = _sc_stage(input_features, q_in.reshape(-1), q_out.reshape(-1),
                    q_ker.reshape(-1))
    return _tc_stage(agg, W.reshape(K * C, C), b)


# speedup vs baseline: 45.0381x; 45.0381x over previous
import functools

import jax
import jax.numpy as jnp
import numpy as np
from jax.experimental import pallas as pl

_THR = 0.15
_BLOCK_O = 256


def _conv_body(x_ref, w_ref, b_ref, qin_bf_ref, qout_ref, qker_ref, out_ref):
    qo = qout_ref[...]
    qk = qker_ref[...]
    qin_bf = qin_bf_ref[...]
    x = x_ref[...]

    ox, oy, oz, ow = qo[:, 0:1], qo[:, 1:2], qo[:, 2:3], qo[:, 3:4]
    cos_half = jnp.float32(np.cos(_THR / 2.0))
    num_k = qk.shape[0]

    acc = jnp.zeros(out_ref.shape, jnp.float32)
    for k in range(num_k):
        kx, ky, kz, kw = qk[k, 0], qk[k, 1], qk[k, 2], qk[k, 3]
        ix = ow * kx + ox * kw + oy * kz - oz * ky
        iy = ow * ky - ox * kz + oy * kw + oz * kx
        iz = ow * kz + ox * ky - oy * kx + oz * kw
        iw = ow * kw - ox * kx - oy * ky - oz * kz
        qik = jnp.concatenate([ix, iy, iz, iw], axis=1)
        dots = jnp.dot(qik.astype(jnp.bfloat16), qin_bf,
                       preferred_element_type=jnp.float32)
        mask = (jnp.abs(dots) > cos_half).astype(jnp.float32)
        counts = jnp.maximum(jnp.sum(mask, axis=1, keepdims=True), 1.0)
        agg = jnp.dot(mask, x, preferred_element_type=jnp.float32) / counts
        acc = acc + jnp.dot(agg, w_ref[k], preferred_element_type=jnp.float32)
    out_ref[...] = acc + b_ref[...]


@functools.partial(jax.jit, static_argnames=())
def kernel(input_features, W, b, q_in, q_out, q_ker):
    n_out = q_out.shape[0]
    n_in, c = input_features.shape
    k, _, d = W.shape
    block_o = min(_BLOCK_O, n_out)
    grid = (n_out // block_o,)
    qin_bf = q_in.T.astype(jnp.bfloat16)

    return pl.pallas_call(
        _conv_body,
        grid=grid,
        in_specs=[
            pl.BlockSpec((n_in, c), lambda i: (0, 0)),
            pl.BlockSpec((k, c, d), lambda i: (0, 0, 0)),
            pl.BlockSpec((1, d), lambda i: (0, 0)),
            pl.BlockSpec((4, n_in), lambda i: (0, 0)),
            pl.BlockSpec((block_o, 4), lambda i: (i, 0)),
            pl.BlockSpec((k, 4), lambda i: (0, 0)),
        ],
        out_specs=pl.BlockSpec((block_o, d), lambda i: (i, 0)),
        out_shape=jax.ShapeDtypeStruct((n_out, d), jnp.float32),
    )(input_features, W, b.reshape(1, d), qin_bf, q_out, q_ker)
